# Initial kernel scaffold; baseline (speedup 1.0000x reference)
import jax
import jax.numpy as jnp
from jax.experimental import pallas as pl


def kernel(x, edge_index, batch, W1, b1, W2, b2, W3, b3, W4, b4, W5, b5, W6, b6, p1, p2):
    raise NotImplementedError("write your pallas kernel here")


# speedup vs baseline: 1.0331x; 1.0331x over previous
import functools
import math

import jax
import jax.numpy as jnp
from jax.experimental import pallas as pl
from jax.experimental.pallas import tpu as pltpu


def _mm_body(a_ref, w_ref, b_ref, o_ref, *, relu):
    acc = jnp.dot(a_ref[...], w_ref[...], preferred_element_type=jnp.float32)
    acc = acc + b_ref[...]
    if relu:
        acc = jnp.maximum(acc, 0.0)
    o_ref[...] = acc


def _mm(a, w, b, relu=False, bm=512):
    m, k = a.shape
    k2, n = w.shape
    grid = (pl.cdiv(m, bm),)
    return pl.pallas_call(
        functools.partial(_mm_body, relu=relu),
        grid=grid,
        in_specs=[
            pl.BlockSpec((bm, k), lambda i: (i, 0)),
            pl.BlockSpec((k, n), lambda i: (0, 0)),
            pl.BlockSpec((1, n), lambda i: (0, 0)),
        ],
        out_specs=pl.BlockSpec((bm, n), lambda i: (i, 0)),
        out_shape=jax.ShapeDtypeStruct((m, n), jnp.float32),
    )(a, w, b.reshape(1, n))


def _gcn(x, src, dst, mask, W, b, n, relu):
    xw = _mm(x, W, b, relu=False)
    deg = jax.ops.segment_sum(mask, dst, num_segments=n) + 1.0
    dinv = 1.0 / jnp.sqrt(deg)
    coef = dinv[src] * dinv[dst] * mask
    out = jax.ops.segment_sum(coef[:, None] * xw[src], dst, num_segments=n)
    out = out + (dinv * dinv)[:, None] * xw
    if relu:
        out = jax.nn.relu(out)
    return out


def _topk_pool(x, src, dst, mask, p, ratio):
    n = x.shape[0]
    score = (x @ p) / jnp.linalg.norm(p)
    k = int(math.ceil(ratio * n))
    vals, perm = jax.lax.top_k(score, k)
    x_new = x[perm] * jnp.tanh(vals)[:, None]
    node_idx = jnp.full((n,), -1, jnp.int32).at[perm].set(jnp.arange(k, dtype=jnp.int32))
    ns = node_idx[src]
    nd = node_idx[dst]
    valid = (ns >= 0) & (nd >= 0)
    new_mask = mask * valid.astype(x.dtype)
    ns = jnp.where(valid, ns, 0)
    nd = jnp.where(valid, nd, 0)
    return x_new, ns, nd, new_mask, perm


def kernel(x, edge_index, batch, W1, b1, W2, b2, W3, b3, W4, b4, W5, b5, W6, b6, p1, p2):
    src = edge_index[0].astype(jnp.int32)
    dst = edge_index[1].astype(jnp.int32)
    n = x.shape[0]
    m0 = jnp.ones((src.shape[0],), x.dtype)
    x1 = _gcn(x, src, dst, m0, W1, b1, n, relu=True)
    x1p, s1, d1, m1, perm1 = _topk_pool(x1, src, dst, m0, p1, 0.5)
    n1 = x1p.shape[0]
    x2 = _gcn(x1p, s1, d1, m1, W2, b2, n1, relu=True)
    x2p, s2, d2, m2, perm2 = _topk_pool(x2, s1, d1, m1, p2, 0.5)
    n2 = x2p.shape[0]
    x3 = _gcn(x2p, s2, d2, m2, W3, b3, n2, relu=True)
    x4 = jnp.zeros((n1, x3.shape[1]), x.dtype).at[perm2].set(x3)
    x4 = _gcn(x4, s1, d1, m1, W4, b4, n1, relu=True)
    x5 = jnp.zeros((n, x4.shape[1]), x.dtype).at[perm1].set(x4)
    x5 = _gcn(x5, src, dst, m0, W5, b5, n, relu=True)
    out = _gcn(x5, src, dst, m0, W6, b6, n, relu=False)
    return out
